# Initial kernel scaffold; baseline (speedup 1.0000x reference)
import jax
import jax.numpy as jnp
from jax.experimental import pallas as pl


def kernel(x, edge_index, W_l, W_r, b_l):
    raise NotImplementedError("write your pallas kernel here")


# speedup vs baseline: 7.5249x; 7.5249x over previous
import functools

import jax
import jax.numpy as jnp
from jax import lax
from jax.experimental import pallas as pl
from jax.experimental.pallas import tpu as pltpu
from jax.experimental.pallas import tpu_sc as plsc

N_NODES = 10000
N_PAD = 10240
D = 128
DP = 144
E = 320000
NC, NS = 2, 16
NW = NC * NS
E_PER_W = E // NW
CHUNK = 80
NCHUNK = E_PER_W // CHUNK
ROWS_PER_TILE = N_PAD // NS


def _sc_aggregate(x_aug, src_r, dst_r, zeros_blk):
    mesh = plsc.VectorSubcoreMesh(core_axis_name="c", subcore_axis_name="s")

    @functools.partial(
        pl.kernel,
        out_type=jax.ShapeDtypeStruct((NC, N_PAD, DP), jnp.float32),
        mesh=mesh,
        compiler_params=pltpu.CompilerParams(use_tc_tiling_on_sc=False),
        scratch_types=[
            pltpu.VMEM_SHARED((N_PAD, DP), jnp.float32),
            pltpu.VMEM((NCHUNK, CHUNK), jnp.int32),
            pltpu.VMEM((NCHUNK, CHUNK), jnp.int32),
            pltpu.VMEM((CHUNK, DP), jnp.float32),
            pltpu.SemaphoreType.DMA,
        ],
    )
    def k(x_hbm, src_hbm, dst_hbm, zeros_hbm, out_hbm,
          acc, src_v, dst_v, rows, sem):
        c = lax.axis_index("c")
        s = lax.axis_index("s")
        w = s * NC + c
        pltpu.sync_copy(zeros_hbm, acc.at[pl.ds(s * ROWS_PER_TILE, ROWS_PER_TILE)])
        pltpu.sync_copy(src_hbm.at[w], src_v)
        pltpu.sync_copy(dst_hbm.at[w], dst_v)
        plsc.subcore_barrier()

        def step(j, carry):
            pltpu.async_copy(x_hbm.at[src_v.at[j]], rows, sem).wait()
            pltpu.sync_copy(rows, acc.at[dst_v.at[j]], add=True)
            return carry

        lax.fori_loop(0, NCHUNK, step, 0)
        plsc.subcore_barrier()
        pltpu.sync_copy(
            acc.at[pl.ds(s * ROWS_PER_TILE, ROWS_PER_TILE)],
            out_hbm.at[c, pl.ds(s * ROWS_PER_TILE, ROWS_PER_TILE)],
        )

    return k(x_aug, src_r, dst_r, zeros_blk)


BLK = 1000


def _tc_tail(parts, x, W_l, W_r, b_l2d):
    def body(p0_ref, p1_ref, x_ref, wl_ref, wr_ref, b_ref, o_ref):
        p = p0_ref[0] + p1_ref[0]
        agg = p[:, :D]
        deg = p[:, D:D + 1]
        mean = agg / jnp.maximum(deg, 1.0)
        h = (jnp.dot(mean, wl_ref[...], preferred_element_type=jnp.float32)
             + b_ref[...]
             + jnp.dot(x_ref[...], wr_ref[...], preferred_element_type=jnp.float32))
        h = jnp.maximum(h, 0.0)
        n = jnp.sqrt(jnp.sum(h * h, axis=1, keepdims=True))
        o_ref[...] = h / (n + 1e-9)

    return pl.pallas_call(
        body,
        grid=(N_NODES // BLK,),
        in_specs=[
            pl.BlockSpec((1, BLK, DP), lambda i: (0, i, 0)),
            pl.BlockSpec((1, BLK, DP), lambda i: (1, i, 0)),
            pl.BlockSpec((BLK, D), lambda i: (i, 0)),
            pl.BlockSpec((D, D), lambda i: (0, 0)),
            pl.BlockSpec((D, D), lambda i: (0, 0)),
            pl.BlockSpec((1, D), lambda i: (0, 0)),
        ],
        out_specs=pl.BlockSpec((BLK, D), lambda i: (i, 0)),
        out_shape=jax.ShapeDtypeStruct((N_NODES, D), jnp.float32),
    )(parts, parts, x, W_l, W_r, b_l2d)


def kernel(x, edge_index, W_l, W_r, b_l):
    ei = edge_index.astype(jnp.int32)
    src_r = ei[0].reshape(NW, NCHUNK, CHUNK)
    dst_r = ei[1].reshape(NW, NCHUNK, CHUNK)
    x_aug = jnp.concatenate(
        [x,
         jnp.ones((N_NODES, 1), jnp.float32),
         jnp.zeros((N_NODES, DP - D - 1), jnp.float32)],
        axis=1,
    )
    zeros_blk = jnp.zeros((ROWS_PER_TILE, DP), jnp.float32)
    parts = _sc_aggregate(x_aug, src_r, dst_r, zeros_blk)
    return _tc_tail(parts, x, W_l, W_r, b_l.reshape(1, D))
